# Initial kernel scaffold; baseline (speedup 1.0000x reference)
import jax
import jax.numpy as jnp
from jax.experimental import pallas as pl


def kernel(short_cat, long_cat, num_features, emb, Wls, bls, Wln, bln, W1, b1, W2, b2, W3, b3, Wo, bo):
    raise NotImplementedError("write your pallas kernel here")


# speedup vs baseline: 11.1420x; 11.1420x over previous
import functools

import jax
import jax.numpy as jnp
from jax import lax
from jax.experimental import pallas as pl
from jax.experimental.pallas import tpu as pltpu
from jax.experimental.pallas import tpu_sc as plsc

B = 4096
F = 26
V = 1000
D = 128
SCD = 143
NUM = 13
H1, H2, H3 = 1024, 512, 256

NC = 2
NS = 16
NW = NC * NS
ROWS = B * F
RPW = ROWS // NW
SROWS = 128
GROUP = 2
GROWS = SROWS * GROUP
NSTREAM = RPW // SROWS
NGROUP = RPW // GROWS


def _sc_gather_body(table_hbm, idx_hbm, out_hbm, idx_v, rows0, rows1, sem0, sem1):
    wid = lax.axis_index("s") * NC + lax.axis_index("c")
    base = wid * RPW
    pltpu.sync_copy(idx_hbm.at[wid], idx_v)
    bufs = (rows0, rows1)
    sems = (sem0, sem1)

    def fire(g):
        buf, sem = bufs[g % 2], sems[g % 2]
        return [pltpu.async_copy(table_hbm.at[idx_v.at[g * GROUP + j]],
                                 buf.at[pl.ds(j * SROWS, SROWS)], sem)
                for j in range(GROUP)]

    pend = [fire(0)]
    for g in range(NGROUP):
        for cp in pend[g]:
            cp.wait()
        if g + 1 < NGROUP:
            pend.append(fire(g + 1))
        pltpu.sync_copy(bufs[g % 2],
                        out_hbm.at[pl.ds(base + g * GROWS, GROWS)])


@functools.cache
def _sc_gather():
    return pl.kernel(
        _sc_gather_body,
        out_type=jax.ShapeDtypeStruct((ROWS, D), jnp.float32),
        mesh=plsc.VectorSubcoreMesh(
            core_axis_name="c", subcore_axis_name="s",
            num_cores=NC, num_subcores=NS),
        scratch_types=[
            pltpu.VMEM((NSTREAM, SROWS), jnp.int32),
            pltpu.VMEM((GROWS, D), jnp.float32),
            pltpu.VMEM((GROWS, D), jnp.float32),
            pltpu.SemaphoreType.DMA,
            pltpu.SemaphoreType.DMA,
        ],
    )


BB = 256
GRID = B // BB


def _tc_body(sc_ref, num_ref, fm_ref, wls_ref, wln_ref,
             w1s_ref, w1n_ref, w1f_ref, b1_ref, w2_ref, b2_ref,
             w3_ref, b3_ref, wo_ref, bias_ref, out_ref):
    fm = fm_ref[...]
    s = fm[:, 0:D]
    sq = s * s
    for f in range(1, F):
        e = fm[:, f * D:(f + 1) * D]
        s = s + e
        sq = sq + e * e
    fm_part = 0.5 * jnp.sum(s * s - sq, axis=1, keepdims=True)

    sc = sc_ref[...]
    nm = num_ref[...]
    linear = sc @ wls_ref[...] + nm @ wln_ref[...]

    h = sc @ w1s_ref[...] + nm @ w1n_ref[...] + fm @ w1f_ref[...] + b1_ref[...]
    h = jnp.maximum(h, 0.0)
    h = jnp.maximum(h @ w2_ref[...] + b2_ref[...], 0.0)
    h = jnp.maximum(h @ w3_ref[...] + b3_ref[...], 0.0)
    out_ref[...] = linear + fm_part + h @ wo_ref[...] + bias_ref[...]


def _full(r, c):
    return pl.BlockSpec((r, c), lambda i: (0, 0))


_tc_fused = pl.pallas_call(
    _tc_body,
    grid=(GRID,),
    in_specs=[
        pl.BlockSpec((BB, SCD), lambda i: (i, 0)),
        pl.BlockSpec((BB, NUM), lambda i: (i, 0)),
        pl.BlockSpec((BB, F * D), lambda i: (i, 0)),
        _full(SCD, 1), _full(NUM, 1),
        _full(SCD, H1), _full(NUM, H1), _full(F * D, H1), _full(1, H1),
        _full(H1, H2), _full(1, H2),
        _full(H2, H3), _full(1, H3),
        _full(H3, 1), _full(1, 1),
    ],
    out_specs=pl.BlockSpec((BB, 1), lambda i: (i, 0)),
    out_shape=jax.ShapeDtypeStruct((B, 1), jnp.float32),
)


def kernel(short_cat, long_cat, num_features, emb, Wls, bls, Wln, bln,
           W1, b1, W2, b2, W3, b3, Wo, bo):
    table = emb.reshape(F * V, D)
    flat_idx = (long_cat.astype(jnp.int32)
                + (jnp.arange(F, dtype=jnp.int32) * V)[None, :])
    idx = flat_idx.reshape(NW, NSTREAM, SROWS)
    fm_rows = _sc_gather()(table, idx)
    fm2d = fm_rows.reshape(B, F * D)
    bias = (bls + bln + bo).reshape(1, 1)
    return _tc_fused(short_cat, num_features, fm2d, Wls, Wln,
                     W1[:SCD], W1[SCD:SCD + NUM], W1[SCD + NUM:],
                     b1.reshape(1, H1), W2, b2.reshape(1, H2),
                     W3, b3.reshape(1, H3), Wo, bias)


# speedup vs baseline: 11.3019x; 1.0144x over previous
import functools

import jax
import jax.numpy as jnp
from jax import lax
from jax.experimental import pallas as pl
from jax.experimental.pallas import tpu as pltpu
from jax.experimental.pallas import tpu_sc as plsc

B = 4096
F = 26
V = 1000
D = 128
SCD = 143
NUM = 13
H1, H2, H3 = 1024, 512, 256

NC = 2
NS = 16
NW = NC * NS
ROWS = B * F
RPW = ROWS // NW
SROWS = 128
GROUP = 2
GROWS = SROWS * GROUP
NSTREAM = RPW // SROWS
NGROUP = RPW // GROWS


def _sc_gather_body(table_hbm, idx_hbm, out_hbm, idx_v, rows0, rows1, sem0, sem1):
    wid = lax.axis_index("s") * NC + lax.axis_index("c")
    base = wid * RPW
    pltpu.sync_copy(idx_hbm.at[wid], idx_v)
    bufs = (rows0, rows1)
    sems = (sem0, sem1)

    def fire(g):
        buf, sem = bufs[g % 2], sems[g % 2]
        return [pltpu.async_copy(table_hbm.at[idx_v.at[g * GROUP + j]],
                                 buf.at[pl.ds(j * SROWS, SROWS)], sem)
                for j in range(GROUP)]

    pend = [fire(0)]
    for g in range(NGROUP):
        for cp in pend[g]:
            cp.wait()
        if g + 1 < NGROUP:
            pend.append(fire(g + 1))
        pltpu.sync_copy(bufs[g % 2],
                        out_hbm.at[pl.ds(base + g * GROWS, GROWS)])


@functools.cache
def _sc_gather():
    return pl.kernel(
        _sc_gather_body,
        out_type=jax.ShapeDtypeStruct((ROWS, D), jnp.float32),
        mesh=plsc.VectorSubcoreMesh(
            core_axis_name="c", subcore_axis_name="s",
            num_cores=NC, num_subcores=NS),
        scratch_types=[
            pltpu.VMEM((NSTREAM, SROWS), jnp.int32),
            pltpu.VMEM((GROWS, D), jnp.float32),
            pltpu.VMEM((GROWS, D), jnp.float32),
            pltpu.SemaphoreType.DMA,
            pltpu.SemaphoreType.DMA,
        ],
    )


BB = 256
GRID = B // BB


def _tc_body(sc_ref, num_ref, fm_ref, wls_ref, wln_ref,
             w1s_ref, w1n_ref, w1f_ref, b1_ref, w2_ref, b2_ref,
             w3_ref, b3_ref, wo_ref, bias_ref, out_ref):
    fm = fm_ref[...]
    s = fm[:, 0:D]
    sq = s * s
    for f in range(1, F):
        e = fm[:, f * D:(f + 1) * D]
        s = s + e
        sq = sq + e * e
    fm_part = 0.5 * jnp.sum(s * s - sq, axis=1, keepdims=True)

    sc = sc_ref[...]
    nm = num_ref[...]
    linear = sc @ wls_ref[...] + nm @ wln_ref[...]

    def mm(a, w):
        return jax.lax.dot(a.astype(jnp.bfloat16), w,
                           preferred_element_type=jnp.float32)

    h = (sc @ w1s_ref[...] + nm @ w1n_ref[...]
         + mm(fm, w1f_ref[...]) + b1_ref[...])
    h = jnp.maximum(h, 0.0)
    h = jnp.maximum(mm(h, w2_ref[...]) + b2_ref[...], 0.0)
    h = jnp.maximum(mm(h, w3_ref[...]) + b3_ref[...], 0.0)
    out_ref[...] = linear + fm_part + mm(h, wo_ref[...]) + bias_ref[...]


def _full(r, c):
    return pl.BlockSpec((r, c), lambda i: (0, 0))


_tc_fused = pl.pallas_call(
    _tc_body,
    grid=(GRID,),
    in_specs=[
        pl.BlockSpec((BB, SCD), lambda i: (i, 0)),
        pl.BlockSpec((BB, NUM), lambda i: (i, 0)),
        pl.BlockSpec((BB, F * D), lambda i: (i, 0)),
        _full(SCD, 1), _full(NUM, 1),
        _full(SCD, H1), _full(NUM, H1), _full(F * D, H1), _full(1, H1),
        _full(H1, H2), _full(1, H2),
        _full(H2, H3), _full(1, H3),
        _full(H3, 1), _full(1, 1),
    ],
    out_specs=pl.BlockSpec((BB, 1), lambda i: (i, 0)),
    out_shape=jax.ShapeDtypeStruct((B, 1), jnp.float32),
)


def kernel(short_cat, long_cat, num_features, emb, Wls, bls, Wln, bln,
           W1, b1, W2, b2, W3, b3, Wo, bo):
    table = emb.reshape(F * V, D)
    flat_idx = (long_cat.astype(jnp.int32)
                + (jnp.arange(F, dtype=jnp.int32) * V)[None, :])
    idx = flat_idx.reshape(NW, NSTREAM, SROWS)
    fm_rows = _sc_gather()(table, idx)
    fm2d = fm_rows.reshape(B, F * D)
    bias = (bls + bln + bo).reshape(1, 1)
    bf = jnp.bfloat16
    return _tc_fused(short_cat, num_features, fm2d, Wls, Wln,
                     W1[:SCD], W1[SCD:SCD + NUM], W1[SCD + NUM:].astype(bf),
                     b1.reshape(1, H1), W2.astype(bf), b2.reshape(1, H2),
                     W3.astype(bf), b3.reshape(1, H3), Wo.astype(bf), bias)


# speedup vs baseline: 15.0739x; 1.3337x over previous
import functools

import jax
import jax.numpy as jnp
from jax import lax
from jax.experimental import pallas as pl
from jax.experimental.pallas import tpu as pltpu
from jax.experimental.pallas import tpu_sc as plsc

B = 4096
F = 26
V = 1000
D = 128
SCD = 143
NUM = 13
H1, H2, H3 = 1024, 512, 256

NC = 2
NS = 16
NW = NC * NS
ROWS = B * F
SROWS = 128
TROWS = 256
NTILE = ROWS // TROWS
TPW = NTILE // NW
BBLK = B // TROWS


def _sc_gather_body(table_hbm, idx_hbm, out_hbm, idx_v, rows0, rows1, sem0, sem1):
    wid = lax.axis_index("s") * NC + lax.axis_index("c")
    t0 = wid * TPW
    pltpu.sync_copy(idx_hbm.at[wid], idx_v)
    bufs = (rows0, rows1)
    sems = (sem0, sem1)

    def fire(k):
        buf, sem = bufs[k % 2], sems[k % 2]
        return [pltpu.async_copy(table_hbm.at[idx_v.at[k, j]],
                                 buf.at[pl.ds(j * SROWS, SROWS)], sem)
                for j in range(2)]

    pend = [fire(0)]
    for k in range(TPW):
        for cp in pend[k]:
            cp.wait()
        if k + 1 < TPW:
            pend.append(fire(k + 1))
        t = t0 + k
        row_base = pl.multiple_of(((t >> 4) << 12) + ((t & 15) << 8), TROWS)
        pltpu.sync_copy(bufs[k % 2], out_hbm.at[pl.ds(row_base, TROWS)])


@functools.cache
def _sc_gather():
    return pl.kernel(
        _sc_gather_body,
        out_type=jax.ShapeDtypeStruct((ROWS, D), jnp.float32),
        mesh=plsc.VectorSubcoreMesh(
            core_axis_name="c", subcore_axis_name="s",
            num_cores=NC, num_subcores=NS),
        scratch_types=[
            pltpu.VMEM((TPW, 2, SROWS), jnp.int32),
            pltpu.VMEM((TROWS, D), jnp.float32),
            pltpu.VMEM((TROWS, D), jnp.float32),
            pltpu.SemaphoreType.DMA,
            pltpu.SemaphoreType.DMA,
        ],
    )


BB = 256
GRID = B // BB


def _tc_body(sc_ref, num_ref, wls_ref, wln_ref, w1_ref, b1_ref,
             w2_ref, b2_ref, w3_ref, b3_ref, wo_ref, bias_ref,
             *rest):
    fm_refs, out_ref = rest[:F], rest[F]
    es = [r[...] for r in fm_refs]

    def tree_sum(vals):
        while len(vals) > 1:
            vals = [a + b for a, b in zip(vals[::2], vals[1::2])] + (
                [vals[-1]] if len(vals) % 2 else [])
        return vals[0]

    s = tree_sum(es)
    sq = tree_sum([e * e for e in es])
    fm_part = 0.5 * jnp.sum(s * s - sq, axis=1, keepdims=True)

    sc = sc_ref[...]
    nm = num_ref[...]
    linear = sc @ wls_ref[...] + nm @ wln_ref[...]

    def mm(a, w):
        return jax.lax.dot(a.astype(jnp.bfloat16), w,
                           preferred_element_type=jnp.float32)

    fm2d = jnp.concatenate(es, axis=1)
    h = (mm(sc, w1_ref[0:SCD]) + mm(nm, w1_ref[SCD:SCD + NUM])
         + mm(fm2d, w1_ref[SCD + NUM:]) + b1_ref[...])
    h = jnp.maximum(h, 0.0)
    h = jnp.maximum(mm(h, w2_ref[...]) + b2_ref[...], 0.0)
    h = jnp.maximum(mm(h, w3_ref[...]) + b3_ref[...], 0.0)
    out_ref[...] = linear + fm_part + mm(h, wo_ref[...]) + bias_ref[...]


def _full(r, c):
    return pl.BlockSpec((r, c), lambda i: (0, 0))


def _fm_spec(f):
    return pl.BlockSpec((BB, D), lambda i, f=f: (f * BBLK + i, 0))


_tc_fused = pl.pallas_call(
    _tc_body,
    grid=(GRID,),
    in_specs=[
        pl.BlockSpec((BB, SCD), lambda i: (i, 0)),
        pl.BlockSpec((BB, NUM), lambda i: (i, 0)),
        _full(SCD, 1), _full(NUM, 1),
        _full(SCD + NUM + F * D, H1), _full(1, H1),
        _full(H1, H2), _full(1, H2),
        _full(H2, H3), _full(1, H3),
        _full(H3, 1), _full(1, 1),
    ] + [_fm_spec(f) for f in range(F)],
    out_specs=pl.BlockSpec((BB, 1), lambda i: (i, 0)),
    out_shape=jax.ShapeDtypeStruct((B, 1), jnp.float32),
)


def kernel(short_cat, long_cat, num_features, emb, Wls, bls, Wln, bln,
           W1, b1, W2, b2, W3, b3, Wo, bo):
    table = emb.reshape(F * V, D)
    flat_idx = (long_cat.astype(jnp.int32)
                + (jnp.arange(F, dtype=jnp.int32) * V)[None, :])
    idx = flat_idx.T.reshape(NTILE, 2, SROWS).reshape(NW, TPW, 2, SROWS)
    fm_rows = _sc_gather()(table, idx)
    bias = (bls + bln + bo).reshape(1, 1)
    bf = jnp.bfloat16
    return _tc_fused(short_cat, num_features, Wls, Wln,
                     W1.astype(bf), b1.reshape(1, H1),
                     W2.astype(bf), b2.reshape(1, H2),
                     W3.astype(bf), b3.reshape(1, H3),
                     Wo.astype(bf), bias,
                     *([fm_rows] * F))


# speedup vs baseline: 15.3566x; 1.0188x over previous
import functools

import jax
import jax.numpy as jnp
from jax import lax
from jax.experimental import pallas as pl
from jax.experimental.pallas import tpu as pltpu
from jax.experimental.pallas import tpu_sc as plsc

B = 4096
F = 26
V = 1000
D = 128
SCD = 143
NUM = 13
H1, H2, H3 = 1024, 512, 256

NC = 2
NS = 16
NW = NC * NS
ROWS = B * F
SROWS = 128
TROWS = 256
NTILE = ROWS // TROWS
TPW = NTILE // NW
BBLK = B // TROWS


def _sc_gather_body(table_hbm, idx_hbm, out_hbm, idx_v, rows0, rows1, sem0, sem1):
    wid = lax.axis_index("s") * NC + lax.axis_index("c")
    t0 = wid * TPW
    pltpu.sync_copy(idx_hbm.at[wid], idx_v)
    bufs = (rows0, rows1)
    sems = (sem0, sem1)

    def fire(k):
        buf, sem = bufs[k % 2], sems[k % 2]
        return [pltpu.async_copy(table_hbm.at[idx_v.at[k, j]],
                                 buf.at[pl.ds(j * SROWS, SROWS)], sem)
                for j in range(2)]

    pend = [fire(0)]
    for k in range(TPW):
        for cp in pend[k]:
            cp.wait()
        if k + 1 < TPW:
            pend.append(fire(k + 1))
        t = t0 + k
        row_base = pl.multiple_of(((t >> 4) << 12) + ((t & 15) << 8), TROWS)
        pltpu.sync_copy(bufs[k % 2], out_hbm.at[pl.ds(row_base, TROWS)])


@functools.cache
def _sc_gather():
    return pl.kernel(
        _sc_gather_body,
        out_type=jax.ShapeDtypeStruct((ROWS, D), jnp.float32),
        mesh=plsc.VectorSubcoreMesh(
            core_axis_name="c", subcore_axis_name="s",
            num_cores=NC, num_subcores=NS),
        scratch_types=[
            pltpu.VMEM((TPW, 2, SROWS), jnp.int32),
            pltpu.VMEM((TROWS, D), jnp.float32),
            pltpu.VMEM((TROWS, D), jnp.float32),
            pltpu.SemaphoreType.DMA,
            pltpu.SemaphoreType.DMA,
        ],
    )


BB = 256
GRID = B // BB


def _tc_body(sc_ref, num_ref, wls_ref, wln_ref, w1_ref, b1_ref,
             w2_ref, b2_ref, w3_ref, b3_ref, wo_ref, bias_ref,
             *rest):
    fm_refs = rest[:F]
    out_ref = rest[F]
    w1b_ref, w2b_ref, w3b_ref, wob_ref = rest[F + 1:F + 5]

    @pl.when(pl.program_id(0) == 0)
    def _cast_weights():
        w1b_ref[...] = w1_ref[...].astype(jnp.bfloat16)
        w2b_ref[...] = w2_ref[...].astype(jnp.bfloat16)
        w3b_ref[...] = w3_ref[...].astype(jnp.bfloat16)
        wob_ref[...] = wo_ref[...].astype(jnp.bfloat16)

    es = [r[...] for r in fm_refs]

    def tree_sum(vals):
        while len(vals) > 1:
            vals = [a + b for a, b in zip(vals[::2], vals[1::2])] + (
                [vals[-1]] if len(vals) % 2 else [])
        return vals[0]

    s = tree_sum(es)
    sq = tree_sum([e * e for e in es])
    fm_part = 0.5 * jnp.sum(s * s - sq, axis=1, keepdims=True)

    sc = sc_ref[...]
    nm = num_ref[...]
    linear = sc @ wls_ref[...] + nm @ wln_ref[...]

    def mm(a, w):
        return jax.lax.dot(a.astype(jnp.bfloat16), w,
                           preferred_element_type=jnp.float32)

    fm2d = jnp.concatenate(es, axis=1)
    h = (mm(sc, w1b_ref[0:SCD]) + mm(nm, w1b_ref[SCD:SCD + NUM])
         + mm(fm2d, w1b_ref[SCD + NUM:]) + b1_ref[...])
    h = jnp.maximum(h, 0.0)
    h = jnp.maximum(mm(h, w2b_ref[...]) + b2_ref[...], 0.0)
    h = jnp.maximum(mm(h, w3b_ref[...]) + b3_ref[...], 0.0)
    out_ref[...] = linear + fm_part + mm(h, wob_ref[...]) + bias_ref[...]


def _full(r, c):
    return pl.BlockSpec((r, c), lambda i: (0, 0))


def _fm_spec(f):
    return pl.BlockSpec((BB, D), lambda i, f=f: (f * BBLK + i, 0))


def _build_tc(interpret=False):
    return pl.pallas_call(
        _tc_body,
        grid=(GRID,),
        in_specs=[
            pl.BlockSpec((BB, SCD), lambda i: (i, 0)),
            pl.BlockSpec((BB, NUM), lambda i: (i, 0)),
            _full(SCD, 1), _full(NUM, 1),
            _full(SCD + NUM + F * D, H1), _full(1, H1),
            _full(H1, H2), _full(1, H2),
            _full(H2, H3), _full(1, H3),
            _full(H3, 1), _full(1, 1),
        ] + [_fm_spec(f) for f in range(F)],
        out_specs=pl.BlockSpec((BB, 1), lambda i: (i, 0)),
        out_shape=jax.ShapeDtypeStruct((B, 1), jnp.float32),
        scratch_shapes=[
            pltpu.VMEM((SCD + NUM + F * D, H1), jnp.bfloat16),
            pltpu.VMEM((H1, H2), jnp.bfloat16),
            pltpu.VMEM((H2, H3), jnp.bfloat16),
            pltpu.VMEM((H3, 1), jnp.bfloat16),
        ],
        interpret=interpret,
    )


_tc_fused = _build_tc()


def kernel(short_cat, long_cat, num_features, emb, Wls, bls, Wln, bln,
           W1, b1, W2, b2, W3, b3, Wo, bo):
    table = emb.reshape(F * V, D)
    flat_idx = (long_cat.astype(jnp.int32)
                + (jnp.arange(F, dtype=jnp.int32) * V)[None, :])
    idx = flat_idx.T.reshape(NTILE, 2, SROWS).reshape(NW, TPW, 2, SROWS)
    fm_rows = _sc_gather()(table, idx)
    bias = (bls + bln + bo).reshape(1, 1)
    return _tc_fused(short_cat, num_features, Wls, Wln,
                     W1, b1.reshape(1, H1),
                     W2, b2.reshape(1, H2),
                     W3, b3.reshape(1, H3),
                     Wo, bias,
                     *([fm_rows] * F))


# speedup vs baseline: 16.0724x; 1.0466x over previous
import functools

import jax
import jax.numpy as jnp
from jax import lax
from jax.experimental import pallas as pl
from jax.experimental.pallas import tpu as pltpu
from jax.experimental.pallas import tpu_sc as plsc

B = 4096
F = 26
V = 1000
D = 128
SCD = 143
NUM = 13
H1, H2, H3 = 1024, 512, 256

NC = 2
NS = 16
NW = NC * NS
SROWS = 128
NSPLIT = 2
NB = B // NSPLIT
TR = NB // 16
SPT = TR // SROWS
NTILE = F * 16
TPW = NTILE // NW
BB = 256
GRID = NB // BB
BBLK = NB // BB


def _sc_gather_body(table_hbm, idx_hbm, out_hbm, idx_v, rows0, rows1, sem0, sem1):
    wid = lax.axis_index("s") * NC + lax.axis_index("c")
    t0 = wid * TPW
    pltpu.sync_copy(idx_hbm.at[wid], idx_v)
    bufs = (rows0, rows1)
    sems = (sem0, sem1)

    def fire(k):
        buf, sem = bufs[k % 2], sems[k % 2]
        return [pltpu.async_copy(table_hbm.at[idx_v.at[k, j]],
                                 buf.at[pl.ds(j * SROWS, SROWS)], sem)
                for j in range(SPT)]

    pend = [fire(0)]
    for k in range(TPW):
        for cp in pend[k]:
            cp.wait()
        if k + 1 < TPW:
            pend.append(fire(k + 1))
        t = t0 + k
        row_base = pl.multiple_of((t >> 4) * NB + (t & 15) * TR, TR)
        pltpu.sync_copy(bufs[k % 2], out_hbm.at[pl.ds(row_base, TR)])


@functools.cache
def _sc_gather():
    return pl.kernel(
        _sc_gather_body,
        out_type=jax.ShapeDtypeStruct((F * NB, D), jnp.float32),
        mesh=plsc.VectorSubcoreMesh(
            core_axis_name="c", subcore_axis_name="s",
            num_cores=NC, num_subcores=NS),
        scratch_types=[
            pltpu.VMEM((TPW, SPT, SROWS), jnp.int32),
            pltpu.VMEM((TR, D), jnp.float32),
            pltpu.VMEM((TR, D), jnp.float32),
            pltpu.SemaphoreType.DMA,
            pltpu.SemaphoreType.DMA,
        ],
    )


def _tc_body(sc_ref, num_ref, wls_ref, wln_ref, w1_ref, b1_ref,
             w2_ref, b2_ref, w3_ref, b3_ref, wo_ref, bias_ref,
             *rest):
    fm_refs = rest[:F]
    out_ref = rest[F]
    w1b_ref, w2b_ref, w3b_ref, wob_ref = rest[F + 1:F + 5]

    @pl.when(pl.program_id(0) == 0)
    def _cast_weights():
        w1b_ref[...] = w1_ref[...].astype(jnp.bfloat16)
        w2b_ref[...] = w2_ref[...].astype(jnp.bfloat16)
        w3b_ref[...] = w3_ref[...].astype(jnp.bfloat16)
        wob_ref[...] = wo_ref[...].astype(jnp.bfloat16)

    es = [r[...] for r in fm_refs]

    def tree_sum(vals):
        while len(vals) > 1:
            vals = [a + b for a, b in zip(vals[::2], vals[1::2])] + (
                [vals[-1]] if len(vals) % 2 else [])
        return vals[0]

    s = tree_sum(es)
    sq = tree_sum([e * e for e in es])
    fm_part = 0.5 * jnp.sum(s * s - sq, axis=1, keepdims=True)

    sc = sc_ref[...]
    nm = num_ref[...]
    linear = sc @ wls_ref[...] + nm @ wln_ref[...]

    def mm(a, w):
        return jax.lax.dot(a.astype(jnp.bfloat16), w,
                           preferred_element_type=jnp.float32)

    fm2d = jnp.concatenate(es, axis=1)
    h = (mm(sc, w1b_ref[0:SCD]) + mm(nm, w1b_ref[SCD:SCD + NUM])
         + mm(fm2d, w1b_ref[SCD + NUM:]) + b1_ref[...])
    h = jnp.maximum(h, 0.0)
    h = jnp.maximum(mm(h, w2b_ref[...]) + b2_ref[...], 0.0)
    h = jnp.maximum(mm(h, w3b_ref[...]) + b3_ref[...], 0.0)
    out_ref[...] = linear + fm_part + mm(h, wob_ref[...]) + bias_ref[...]


def _full(r, c):
    return pl.BlockSpec((r, c), lambda i: (0, 0))


def _fm_spec(f):
    return pl.BlockSpec((BB, D), lambda i, f=f: (f * BBLK + i, 0))


def _build_tc(interpret=False):
    return pl.pallas_call(
        _tc_body,
        grid=(GRID,),
        in_specs=[
            pl.BlockSpec((BB, SCD), lambda i: (i, 0)),
            pl.BlockSpec((BB, NUM), lambda i: (i, 0)),
            _full(SCD, 1), _full(NUM, 1),
            _full(SCD + NUM + F * D, H1), _full(1, H1),
            _full(H1, H2), _full(1, H2),
            _full(H2, H3), _full(1, H3),
            _full(H3, 1), _full(1, 1),
        ] + [_fm_spec(f) for f in range(F)],
        out_specs=pl.BlockSpec((BB, 1), lambda i: (i, 0)),
        out_shape=jax.ShapeDtypeStruct((NB, 1), jnp.float32),
        scratch_shapes=[
            pltpu.VMEM((SCD + NUM + F * D, H1), jnp.bfloat16),
            pltpu.VMEM((H1, H2), jnp.bfloat16),
            pltpu.VMEM((H2, H3), jnp.bfloat16),
            pltpu.VMEM((H3, 1), jnp.bfloat16),
        ],
        interpret=interpret,
    )


_tc_fused = _build_tc()


def kernel(short_cat, long_cat, num_features, emb, Wls, bls, Wln, bln,
           W1, b1, W2, b2, W3, b3, Wo, bo):
    table = emb.reshape(F * V, D)
    flat_idx = (long_cat.astype(jnp.int32)
                + (jnp.arange(F, dtype=jnp.int32) * V)[None, :])
    bias = (bls + bln + bo).reshape(1, 1)
    b1r, b2r, b3r = b1.reshape(1, H1), b2.reshape(1, H2), b3.reshape(1, H3)
    outs = []
    for hseg in range(NSPLIT):
        lo = hseg * NB
        idx = (flat_idx[lo:lo + NB].T
               .reshape(NTILE, SPT, SROWS).reshape(NW, TPW, SPT, SROWS))
        fm_rows = _sc_gather()(table, idx)
        outs.append(_tc_fused(
            short_cat[lo:lo + NB], num_features[lo:lo + NB], Wls, Wln,
            W1, b1r, W2, b2r, W3, b3r, Wo, bias, *([fm_rows] * F)))
    return jnp.concatenate(outs, axis=0)


# speedup vs baseline: 17.1073x; 1.0644x over previous
import functools

import jax
import jax.numpy as jnp
from jax import lax
from jax.experimental import pallas as pl
from jax.experimental.pallas import tpu as pltpu
from jax.experimental.pallas import tpu_sc as plsc

B = 4096
F = 26
V = 1000
D = 128
SCD = 143
NUM = 13
H1, H2, H3 = 1024, 512, 256

NC = 2
NS = 16
NW = NC * NS
SROWS = 128
NSPLIT = 2
NB = B // NSPLIT
TR = NB // 16
SPT = TR // SROWS
NTILE = F * 16
TPW = NTILE // NW
BB = 512
GRID = NB // BB
BBLK = NB // BB


def _sc_gather_body(table_hbm, idx_hbm, out_hbm, idx_v, rows0, rows1, sem0, sem1):
    wid = lax.axis_index("s") * NC + lax.axis_index("c")
    t0 = wid * TPW
    pltpu.sync_copy(idx_hbm.at[wid], idx_v)
    bufs = (rows0, rows1)
    sems = (sem0, sem1)

    def fire(k):
        buf, sem = bufs[k % 2], sems[k % 2]
        return [pltpu.async_copy(table_hbm.at[idx_v.at[k, j]],
                                 buf.at[pl.ds(j * SROWS, SROWS)], sem)
                for j in range(SPT)]

    pend = [fire(0)]
    for k in range(TPW):
        for cp in pend[k]:
            cp.wait()
        if k + 1 < TPW:
            pend.append(fire(k + 1))
        t = t0 + k
        row_base = pl.multiple_of((t >> 4) * NB + (t & 15) * TR, TR)
        pltpu.sync_copy(bufs[k % 2], out_hbm.at[pl.ds(row_base, TR)])


@functools.cache
def _sc_gather():
    return pl.kernel(
        _sc_gather_body,
        out_type=jax.ShapeDtypeStruct((F * NB, D), jnp.float32),
        mesh=plsc.VectorSubcoreMesh(
            core_axis_name="c", subcore_axis_name="s",
            num_cores=NC, num_subcores=NS),
        scratch_types=[
            pltpu.VMEM((TPW, SPT, SROWS), jnp.int32),
            pltpu.VMEM((TR, D), jnp.float32),
            pltpu.VMEM((TR, D), jnp.float32),
            pltpu.SemaphoreType.DMA,
            pltpu.SemaphoreType.DMA,
        ],
    )


def _tc_body(sc_ref, num_ref, wls_ref, wln_ref, w1_ref, b1_ref,
             w2_ref, b2_ref, w3_ref, b3_ref, wo_ref, bias_ref,
             *rest):
    fm_refs = rest[:F]
    out_ref = rest[F]
    w1b_ref, w2b_ref, w3b_ref, wob_ref = w1_ref, w2_ref, w3_ref, wo_ref

    es = [r[...] for r in fm_refs]

    def tree_sum(vals):
        while len(vals) > 1:
            vals = [a + b for a, b in zip(vals[::2], vals[1::2])] + (
                [vals[-1]] if len(vals) % 2 else [])
        return vals[0]

    s = tree_sum(es)
    sq = tree_sum([e * e for e in es])
    fm_part = 0.5 * jnp.sum(s * s - sq, axis=1, keepdims=True)

    sc = sc_ref[...]
    nm = num_ref[...]
    linear = sc @ wls_ref[...] + nm @ wln_ref[...]

    def mm(a, w):
        return jax.lax.dot(a.astype(jnp.bfloat16), w,
                           preferred_element_type=jnp.float32)

    fm2d = jnp.concatenate(es, axis=1)
    h = (mm(sc, w1b_ref[0:SCD]) + mm(nm, w1b_ref[SCD:SCD + NUM])
         + mm(fm2d, w1b_ref[SCD + NUM:]) + b1_ref[...])
    h = jnp.maximum(h, 0.0)
    h = jnp.maximum(mm(h, w2b_ref[...]) + b2_ref[...], 0.0)
    h = jnp.maximum(mm(h, w3b_ref[...]) + b3_ref[...], 0.0)
    out_ref[...] = linear + fm_part + mm(h, wob_ref[...]) + bias_ref[...]


def _full(r, c):
    return pl.BlockSpec((r, c), lambda i: (0, 0))


def _fm_spec(f):
    return pl.BlockSpec((BB, D), lambda i, f=f: (f * BBLK + i, 0))


def _build_tc(interpret=False):
    return pl.pallas_call(
        _tc_body,
        grid=(GRID,),
        in_specs=[
            pl.BlockSpec((BB, SCD), lambda i: (i, 0)),
            pl.BlockSpec((BB, NUM), lambda i: (i, 0)),
            _full(SCD, 1), _full(NUM, 1),
            _full(SCD + NUM + F * D, H1), _full(1, H1),
            _full(H1, H2), _full(1, H2),
            _full(H2, H3), _full(1, H3),
            _full(H3, 1), _full(1, 1),
        ] + [_fm_spec(f) for f in range(F)],
        out_specs=pl.BlockSpec((BB, 1), lambda i: (i, 0)),
        out_shape=jax.ShapeDtypeStruct((NB, 1), jnp.float32),
        interpret=interpret,
    )


_tc_fused = _build_tc()


def kernel(short_cat, long_cat, num_features, emb, Wls, bls, Wln, bln,
           W1, b1, W2, b2, W3, b3, Wo, bo):
    table = emb.reshape(F * V, D)
    flat_idx = (long_cat.astype(jnp.int32)
                + (jnp.arange(F, dtype=jnp.int32) * V)[None, :])
    bias = (bls + bln + bo).reshape(1, 1)
    bf = jnp.bfloat16
    W1b, W2b, W3b, Wob = W1.astype(bf), W2.astype(bf), W3.astype(bf), Wo.astype(bf)
    b1r, b2r, b3r = b1.reshape(1, H1), b2.reshape(1, H2), b3.reshape(1, H3)
    outs = []
    for hseg in range(NSPLIT):
        lo = hseg * NB
        idx = (flat_idx[lo:lo + NB].T
               .reshape(NTILE, SPT, SROWS).reshape(NW, TPW, SPT, SROWS))
        fm_rows = _sc_gather()(table, idx)
        outs.append(_tc_fused(
            short_cat[lo:lo + NB], num_features[lo:lo + NB], Wls, Wln,
            W1b, b1r, W2b, b2r, W3b, b3r, Wob, bias, *([fm_rows] * F)))
    return jnp.concatenate(outs, axis=0)
